# baseline (speedup 1.0000x reference)
import jax
import jax.numpy as jnp
from jax import lax
from jax.experimental import pallas as pl
from jax.experimental.pallas import tpu as pltpu
from jax.experimental.pallas import tpu_sc as plsc

B = 16384
D = 64
NE = 1000000
NR = 1000
W = 2 * D
L = 16
NC = 2
NS = 16
NW = NC * NS
BPW = B // NW
C = 16
NCHUNK = BPW // C


def _transe_body(e1p_hbm, e1o_hbm, e2p_hbm, e2o_hbm, rlb_hbm,
                 ent_hbm, relf_hbm, gam_hbm,
                 out_hbm,
                 e1p_v, e1o_v, e2p_v, e2o_v, rlb_v,
                 rel_v, head_v, tail_v, out_v, gam_v,
                 sem1, sem2, sem3):
    wid = lax.axis_index("s") * NC + lax.axis_index("c")
    base = wid * BPW

    pltpu.sync_copy(e1p_hbm.at[pl.ds(base, BPW)], e1p_v)
    pltpu.sync_copy(e2p_hbm.at[pl.ds(base, BPW)], e2p_v)
    pltpu.sync_copy(e1o_hbm.at[pl.ds(base, BPW)], e1o_v)
    pltpu.sync_copy(e2o_hbm.at[pl.ds(base, BPW)], e2o_v)
    pltpu.sync_copy(rlb_hbm.at[pl.ds(base, BPW)], rlb_v)
    pltpu.sync_copy(gam_hbm, gam_v)
    cpr = pltpu.async_copy(relf_hbm, rel_v, sem3)

    gam = gam_v[...]
    j = lax.iota(jnp.int32, L)
    cpr.wait()

    def chunk_body(k, carry):
        off = k * C
        cp1 = pltpu.async_copy(ent_hbm.at[e1p_v.at[pl.ds(off, C)]], head_v,
                               sem1)
        cp2 = pltpu.async_copy(ent_hbm.at[e2p_v.at[pl.ds(off, C)]], tail_v,
                               sem2)
        cp1.wait()
        cp2.wait()

        o1 = e1o_v[pl.ds(off, L)]
        o2 = e2o_v[pl.ds(off, L)]
        rb = rlb_v[pl.ds(off, L)]
        acc = jnp.zeros((L,), jnp.float32)
        for d in range(D):
            h = plsc.load_gather(head_v, [j, o1 + d])
            t = plsc.load_gather(tail_v, [j, o2 + d])
            r = plsc.load_gather(rel_v, [rb + d])
            acc = acc + jnp.abs(h + r - t)
        score = gam - acc
        out_v[pl.ds(off, L)] = 1.0 / (1.0 + jnp.exp(-score))
        return carry

    lax.fori_loop(0, NCHUNK, chunk_body, 0)

    pltpu.sync_copy(out_v, out_hbm.at[pl.ds(base, BPW)])


@jax.jit
def _transe_call(e1p, e1o, e2p, e2o, rlb, ent2, relf, gam_vec):
    mesh = plsc.VectorSubcoreMesh(core_axis_name="c", subcore_axis_name="s")
    f = pl.kernel(
        _transe_body,
        mesh=mesh,
        compiler_params=pltpu.CompilerParams(needs_layout_passes=False),
        out_type=jax.ShapeDtypeStruct((B,), jnp.float32),
        scratch_types=[
            pltpu.VMEM((BPW,), jnp.int32),
            pltpu.VMEM((BPW,), jnp.int32),
            pltpu.VMEM((BPW,), jnp.int32),
            pltpu.VMEM((BPW,), jnp.int32),
            pltpu.VMEM((BPW,), jnp.int32),
            pltpu.VMEM((NR * D,), jnp.float32),
            pltpu.VMEM((C, W), jnp.float32),
            pltpu.VMEM((C, W), jnp.float32),
            pltpu.VMEM((BPW,), jnp.float32),
            pltpu.VMEM((L,), jnp.float32),
            pltpu.SemaphoreType.DMA,
            pltpu.SemaphoreType.DMA,
            pltpu.SemaphoreType.DMA,
        ],
    )
    return f(e1p, e1o, e2p, e2o, rlb, ent2, relf, gam_vec)


def kernel(e1_idx, rel_idx, e2_idx, emb_ent_real, emb_rel_real, gamma):
    e1 = e1_idx.astype(jnp.int32)
    e2 = e2_idx.astype(jnp.int32)
    rlb = rel_idx.astype(jnp.int32) * D
    ent2 = emb_ent_real.reshape(NE // 2, W)
    relf = emb_rel_real.reshape(NR * D)
    gam_vec = jnp.full((L,), gamma, jnp.float32)
    return _transe_call(e1 >> 1, (e1 & 1) * D, e2 >> 1, (e2 & 1) * D, rlb,
                        ent2, relf, gam_vec)


# speedup vs baseline: 3.4868x; 3.4868x over previous
import jax
import jax.numpy as jnp
from jax import lax
from jax.experimental import pallas as pl
from jax.experimental.pallas import tpu as pltpu
from jax.experimental.pallas import tpu_sc as plsc

B = 16384
D = 64
NE = 1000000
NR = 1000
L = 16
NC = 2
NS = 16
NW = NC * NS
TPT = B // NS
PPC = D // NC
RND = 256


def _plane_body(e1_hbm, e2_hbm, rl_hbm,
                entT_hbm, relT_hbm,
                part_hbm,
                e1_v, e2_v, rl_v,
                rel_v, hv, tv, acc_v, spm0, spm1,
                ssem, gsem1, gsem2, rsem):
    cid = lax.axis_index("c")
    sid = lax.axis_index("s")
    tbase = sid * TPT
    c0 = cid * PPC

    pltpu.sync_copy(e1_hbm.at[pl.ds(tbase, TPT)], e1_v)
    pltpu.sync_copy(e2_hbm.at[pl.ds(tbase, TPT)], e2_v)
    pltpu.sync_copy(rl_hbm.at[pl.ds(tbase, TPT)], rl_v)

    for g in range(TPT // L):
        acc_v[pl.ds(g * L, L)] = jnp.zeros((L,), jnp.float32)

    @pl.when(sid == 0)
    def _():
        pltpu.async_copy(entT_hbm.at[c0], spm0, ssem)
        pltpu.async_copy(entT_hbm.at[c0 + 1], spm1, ssem)

    def plane_iter(k, carry):
        for par in range(2):
            c = c0 + 2 * k + par
            spm = spm0 if par == 0 else spm1
            @pl.when(sid == 0)
            def _():
                pltpu.make_async_copy(
                    entT_hbm.at[c], spm, ssem).wait()
            plsc.subcore_barrier()

            pltpu.sync_copy(relT_hbm.at[c], rel_v)
            for rnd in range(TPT // RND):
                roff = rnd * RND
                cp1 = pltpu.async_copy(
                    spm.at[e1_v.at[pl.ds(roff, RND)]], hv, gsem1)
                cp2 = pltpu.async_copy(
                    spm.at[e2_v.at[pl.ds(roff, RND)]], tv, gsem2)
                cp1.wait()
                cp2.wait()
                for g in range(RND // L):
                    lsl = pl.ds(g * L, L)
                    sl = pl.ds(roff + g * L, L)
                    h = hv[lsl]
                    t = tv[lsl]
                    r = plsc.load_gather(rel_v, [rl_v[sl]])
                    acc_v[sl] = acc_v[sl] + jnp.abs(h + r - t)
            plsc.subcore_barrier()

            @pl.when(jnp.logical_and(sid == 0, k < PPC // 2 - 1))
            def _():
                pltpu.async_copy(entT_hbm.at[c + 2], spm, ssem)
        return carry

    lax.fori_loop(0, PPC // 2, plane_iter, 0)

    pltpu.sync_copy(acc_v, part_hbm.at[pl.ds(cid * B + tbase, TPT)])


def _sig_body(part_hbm, gam_hbm, out_hbm, p0_v, p1_v, out_v, gam_v):
    wid = lax.axis_index("s") * NC + lax.axis_index("c")
    base = wid * (B // NW)
    n = B // NW
    pltpu.sync_copy(part_hbm.at[pl.ds(base, n)], p0_v)
    pltpu.sync_copy(part_hbm.at[pl.ds(B + base, n)], p1_v)
    pltpu.sync_copy(gam_hbm, gam_v)
    gam = gam_v[...]
    for g in range(n // L):
        sl = pl.ds(g * L, L)
        score = gam - p0_v[sl] - p1_v[sl]
        out_v[sl] = 1.0 / (1.0 + jnp.exp(-score))
    pltpu.sync_copy(out_v, out_hbm.at[pl.ds(base, n)])


@jax.jit
def _transe_call(e1, e2, rl, entT, relT, gam_vec):
    mesh = plsc.VectorSubcoreMesh(core_axis_name="c", subcore_axis_name="s")
    fa = pl.kernel(
        _plane_body,
        mesh=mesh,
        compiler_params=pltpu.CompilerParams(
            needs_layout_passes=False, internal_scratch_in_bytes=0),
        out_type=jax.ShapeDtypeStruct((2 * B,), jnp.float32),
        scratch_types=[
            pltpu.VMEM((TPT,), jnp.int32),
            pltpu.VMEM((TPT,), jnp.int32),
            pltpu.VMEM((TPT,), jnp.int32),
            pltpu.VMEM((NR,), jnp.float32),
            pltpu.VMEM((RND,), jnp.float32),
            pltpu.VMEM((RND,), jnp.float32),
            pltpu.VMEM((TPT,), jnp.float32),
            pltpu.VMEM_SHARED((NE,), jnp.float32),
            pltpu.VMEM_SHARED((NE,), jnp.float32),
            pltpu.SemaphoreType.DMA,
            pltpu.SemaphoreType.DMA,
            pltpu.SemaphoreType.DMA,
            pltpu.SemaphoreType.DMA,
        ],
    )
    part = fa(e1, e2, rl, entT, relT)
    fb = pl.kernel(
        _sig_body,
        mesh=mesh,
        compiler_params=pltpu.CompilerParams(needs_layout_passes=False),
        out_type=jax.ShapeDtypeStruct((B,), jnp.float32),
        scratch_types=[
            pltpu.VMEM((B // NW,), jnp.float32),
            pltpu.VMEM((B // NW,), jnp.float32),
            pltpu.VMEM((B // NW,), jnp.float32),
            pltpu.VMEM((L,), jnp.float32),
        ],
    )
    return fb(part, gam_vec)


def kernel(e1_idx, rel_idx, e2_idx, emb_ent_real, emb_rel_real, gamma):
    e1 = e1_idx.astype(jnp.int32)
    e2 = e2_idx.astype(jnp.int32)
    rl = rel_idx.astype(jnp.int32)
    gam_vec = jnp.full((L,), gamma, jnp.float32)
    return _transe_call(e1, e2, rl, emb_ent_real.T, emb_rel_real.T, gam_vec)


# speedup vs baseline: 3.6472x; 1.0460x over previous
import jax
import jax.numpy as jnp
from jax import lax
from jax.experimental import pallas as pl
from jax.experimental.pallas import tpu as pltpu
from jax.experimental.pallas import tpu_sc as plsc

B = 16384
D = 64
NE = 1000000
NR = 1000
L = 16
NC = 2
NS = 16
NW = NC * NS
TPT = B // NS
PPC = D // NC
RND = 128


def _plane_body(e1_hbm, e2_hbm, rl_hbm,
                entT_hbm, relT_hbm,
                part_hbm,
                e1_v, e2_v, rl_v,
                rel_v, hv, tv, acc_v, spm0, spm1,
                ssem, gsem1, gsem2, rsem):
    cid = lax.axis_index("c")
    sid = lax.axis_index("s")
    tbase = sid * TPT
    c0 = cid * PPC

    pltpu.sync_copy(e1_hbm.at[pl.ds(tbase, TPT)], e1_v)
    pltpu.sync_copy(e2_hbm.at[pl.ds(tbase, TPT)], e2_v)
    pltpu.sync_copy(rl_hbm.at[pl.ds(tbase, TPT)], rl_v)

    for g in range(TPT // L):
        acc_v[pl.ds(g * L, L)] = jnp.zeros((L,), jnp.float32)

    @pl.when(sid == 0)
    def _():
        pltpu.async_copy(entT_hbm.at[c0], spm0, ssem)
        pltpu.async_copy(entT_hbm.at[c0 + 1], spm1, ssem)

    def plane_iter(k, carry):
        for par in range(2):
            c = c0 + 2 * k + par
            spm = spm0 if par == 0 else spm1
            @pl.when(sid == 0)
            def _():
                pltpu.make_async_copy(
                    entT_hbm.at[c], spm, ssem).wait()
            plsc.subcore_barrier()

            nr = TPT // RND
            pltpu.async_copy(spm.at[e1_v.at[pl.ds(0, RND)]], hv[0], gsem1)
            pltpu.async_copy(spm.at[e2_v.at[pl.ds(0, RND)]], tv[0], gsem2)
            pltpu.sync_copy(relT_hbm.at[c], rel_v)
            for rnd in range(nr):
                cur = rnd % 2
                if rnd + 1 < nr:
                    roff = (rnd + 1) * RND
                    pltpu.async_copy(
                        spm.at[e1_v.at[pl.ds(roff, RND)]], hv[1 - cur],
                        gsem1)
                    pltpu.async_copy(
                        spm.at[e2_v.at[pl.ds(roff, RND)]], tv[1 - cur],
                        gsem2)
                pltpu.make_async_copy(
                    spm.at[e1_v.at[pl.ds(rnd * RND, RND)]], hv[cur],
                    gsem1).wait()
                pltpu.make_async_copy(
                    spm.at[e2_v.at[pl.ds(rnd * RND, RND)]], tv[cur],
                    gsem2).wait()
                for g in range(RND // L):
                    lsl = pl.ds(g * L, L)
                    sl = pl.ds(rnd * RND + g * L, L)
                    h = hv[cur][lsl]
                    t = tv[cur][lsl]
                    r = plsc.load_gather(rel_v, [rl_v[sl]])
                    acc_v[sl] = acc_v[sl] + jnp.abs(h + r - t)
            plsc.subcore_barrier()

            @pl.when(jnp.logical_and(sid == 0, k < PPC // 2 - 1))
            def _():
                pltpu.async_copy(entT_hbm.at[c + 2], spm, ssem)
        return carry

    lax.fori_loop(0, PPC // 2, plane_iter, 0)

    pltpu.sync_copy(acc_v, part_hbm.at[pl.ds(cid * B + tbase, TPT)])


def _sig_body(part_hbm, gam_hbm, out_hbm, p0_v, p1_v, out_v, gam_v):
    wid = lax.axis_index("s") * NC + lax.axis_index("c")
    base = wid * (B // NW)
    n = B // NW
    pltpu.sync_copy(part_hbm.at[pl.ds(base, n)], p0_v)
    pltpu.sync_copy(part_hbm.at[pl.ds(B + base, n)], p1_v)
    pltpu.sync_copy(gam_hbm, gam_v)
    gam = gam_v[...]
    for g in range(n // L):
        sl = pl.ds(g * L, L)
        score = gam - p0_v[sl] - p1_v[sl]
        out_v[sl] = 1.0 / (1.0 + jnp.exp(-score))
    pltpu.sync_copy(out_v, out_hbm.at[pl.ds(base, n)])


@jax.jit
def _transe_call(e1, e2, rl, entT, relT, gam_vec):
    mesh = plsc.VectorSubcoreMesh(core_axis_name="c", subcore_axis_name="s")
    fa = pl.kernel(
        _plane_body,
        mesh=mesh,
        compiler_params=pltpu.CompilerParams(
            needs_layout_passes=False, internal_scratch_in_bytes=0),
        out_type=jax.ShapeDtypeStruct((2 * B,), jnp.float32),
        scratch_types=[
            pltpu.VMEM((TPT,), jnp.int32),
            pltpu.VMEM((TPT,), jnp.int32),
            pltpu.VMEM((TPT,), jnp.int32),
            pltpu.VMEM((NR,), jnp.float32),
            [pltpu.VMEM((RND,), jnp.float32),
             pltpu.VMEM((RND,), jnp.float32)],
            [pltpu.VMEM((RND,), jnp.float32),
             pltpu.VMEM((RND,), jnp.float32)],
            pltpu.VMEM((TPT,), jnp.float32),
            pltpu.VMEM_SHARED((NE,), jnp.float32),
            pltpu.VMEM_SHARED((NE,), jnp.float32),
            pltpu.SemaphoreType.DMA,
            pltpu.SemaphoreType.DMA,
            pltpu.SemaphoreType.DMA,
            pltpu.SemaphoreType.DMA,
        ],
    )
    part = fa(e1, e2, rl, entT, relT)
    fb = pl.kernel(
        _sig_body,
        mesh=mesh,
        compiler_params=pltpu.CompilerParams(needs_layout_passes=False),
        out_type=jax.ShapeDtypeStruct((B,), jnp.float32),
        scratch_types=[
            pltpu.VMEM((B // NW,), jnp.float32),
            pltpu.VMEM((B // NW,), jnp.float32),
            pltpu.VMEM((B // NW,), jnp.float32),
            pltpu.VMEM((L,), jnp.float32),
        ],
    )
    return fb(part, gam_vec)


def kernel(e1_idx, rel_idx, e2_idx, emb_ent_real, emb_rel_real, gamma):
    e1 = e1_idx.astype(jnp.int32)
    e2 = e2_idx.astype(jnp.int32)
    rl = rel_idx.astype(jnp.int32)
    gam_vec = jnp.full((L,), gamma, jnp.float32)
    return _transe_call(e1, e2, rl, emb_ent_real.T, emb_rel_real.T, gam_vec)


# speedup vs baseline: 3.6698x; 1.0062x over previous
import jax
import jax.numpy as jnp
from jax import lax
from jax.experimental import pallas as pl
from jax.experimental.pallas import tpu as pltpu
from jax.experimental.pallas import tpu_sc as plsc

B = 16384
D = 64
NE = 1000000
NR = 1000
L = 16
NC = 2
NS = 16
NW = NC * NS
TPT = B // NS
PPC = D // NC
RND = 128


def _plane_body(e12_hbm, rl_hbm,
                entT_hbm, relT_hbm,
                part_hbm,
                e12_v, rl_v,
                rel_v, htv, acc_v, spm0, spm1,
                ssem, gsem1, gsem2, rsem):
    cid = lax.axis_index("c")
    sid = lax.axis_index("s")
    tbase = sid * TPT
    c0 = cid * PPC

    pltpu.sync_copy(e12_hbm.at[pl.ds(2 * tbase, 2 * TPT)], e12_v)
    pltpu.sync_copy(rl_hbm.at[pl.ds(tbase, TPT)], rl_v)

    for g in range(TPT // L):
        acc_v[pl.ds(g * L, L)] = jnp.zeros((L,), jnp.float32)

    @pl.when(sid == 0)
    def _():
        pltpu.async_copy(entT_hbm.at[c0], spm0, ssem)
        pltpu.async_copy(entT_hbm.at[c0 + 1], spm1, ssem)

    def plane_iter(k, carry):
        for par in range(2):
            c = c0 + 2 * k + par
            spm = spm0 if par == 0 else spm1
            @pl.when(sid == 0)
            def _():
                pltpu.make_async_copy(
                    entT_hbm.at[c], spm, ssem).wait()
            plsc.subcore_barrier()

            nr = TPT // RND
            pltpu.async_copy(
                spm.at[e12_v.at[pl.ds(0, 2 * RND)]], htv[0], gsem1)
            pltpu.sync_copy(relT_hbm.at[c], rel_v)
            for rnd in range(nr):
                cur = rnd % 2
                if rnd + 1 < nr:
                    roff = (rnd + 1) * 2 * RND
                    pltpu.async_copy(
                        spm.at[e12_v.at[pl.ds(roff, 2 * RND)]],
                        htv[1 - cur], gsem1)
                pltpu.make_async_copy(
                    spm.at[e12_v.at[pl.ds(rnd * 2 * RND, 2 * RND)]],
                    htv[cur], gsem1).wait()
                for g in range(RND // L):
                    hsl = pl.ds(g * L, L)
                    tsl = pl.ds(RND + g * L, L)
                    sl = pl.ds(rnd * RND + g * L, L)
                    h = htv[cur][hsl]
                    t = htv[cur][tsl]
                    r = plsc.load_gather(rel_v, [rl_v[sl]])
                    acc_v[sl] = acc_v[sl] + jnp.abs(h + r - t)
            plsc.subcore_barrier()

            @pl.when(jnp.logical_and(sid == 0, k < PPC // 2 - 1))
            def _():
                pltpu.async_copy(entT_hbm.at[c + 2], spm, ssem)
        return carry

    lax.fori_loop(0, PPC // 2, plane_iter, 0)

    pltpu.sync_copy(acc_v, part_hbm.at[pl.ds(cid * B + tbase, TPT)])


def _sig_body(part_hbm, gam_hbm, out_hbm, p0_v, p1_v, out_v, gam_v):
    wid = lax.axis_index("s") * NC + lax.axis_index("c")
    base = wid * (B // NW)
    n = B // NW
    pltpu.sync_copy(part_hbm.at[pl.ds(base, n)], p0_v)
    pltpu.sync_copy(part_hbm.at[pl.ds(B + base, n)], p1_v)
    pltpu.sync_copy(gam_hbm, gam_v)
    gam = gam_v[...]
    for g in range(n // L):
        sl = pl.ds(g * L, L)
        score = gam - p0_v[sl] - p1_v[sl]
        out_v[sl] = 1.0 / (1.0 + jnp.exp(-score))
    pltpu.sync_copy(out_v, out_hbm.at[pl.ds(base, n)])


@jax.jit
def _transe_call(e12, rl, entT, relT, gam_vec):
    mesh = plsc.VectorSubcoreMesh(core_axis_name="c", subcore_axis_name="s")
    fa = pl.kernel(
        _plane_body,
        mesh=mesh,
        compiler_params=pltpu.CompilerParams(
            needs_layout_passes=False, internal_scratch_in_bytes=0),
        out_type=jax.ShapeDtypeStruct((2 * B,), jnp.float32),
        scratch_types=[
            pltpu.VMEM((2 * TPT,), jnp.int32),
            pltpu.VMEM((TPT,), jnp.int32),
            pltpu.VMEM((NR,), jnp.float32),
            [pltpu.VMEM((2 * RND,), jnp.float32),
             pltpu.VMEM((2 * RND,), jnp.float32)],
            pltpu.VMEM((TPT,), jnp.float32),
            pltpu.VMEM_SHARED((NE,), jnp.float32),
            pltpu.VMEM_SHARED((NE,), jnp.float32),
            pltpu.SemaphoreType.DMA,
            pltpu.SemaphoreType.DMA,
            pltpu.SemaphoreType.DMA,
            pltpu.SemaphoreType.DMA,
        ],
    )
    part = fa(e12, rl, entT, relT)
    fb = pl.kernel(
        _sig_body,
        mesh=mesh,
        compiler_params=pltpu.CompilerParams(needs_layout_passes=False),
        out_type=jax.ShapeDtypeStruct((B,), jnp.float32),
        scratch_types=[
            pltpu.VMEM((B // NW,), jnp.float32),
            pltpu.VMEM((B // NW,), jnp.float32),
            pltpu.VMEM((B // NW,), jnp.float32),
            pltpu.VMEM((L,), jnp.float32),
        ],
    )
    return fb(part, gam_vec)


def kernel(e1_idx, rel_idx, e2_idx, emb_ent_real, emb_rel_real, gamma):
    e1 = e1_idx.astype(jnp.int32)
    e2 = e2_idx.astype(jnp.int32)
    rl = rel_idx.astype(jnp.int32)
    e1r = e1.reshape(NS, TPT // RND, 1, RND)
    e2r = e2.reshape(NS, TPT // RND, 1, RND)
    e12 = jnp.concatenate([e1r, e2r], axis=2).reshape(2 * B)
    gam_vec = jnp.full((L,), gamma, jnp.float32)
    return _transe_call(e12, rl, emb_ent_real.T, emb_rel_real.T, gam_vec)


# speedup vs baseline: 3.9121x; 1.0660x over previous
import jax
import jax.numpy as jnp
from jax import lax
from jax.experimental import pallas as pl
from jax.experimental.pallas import tpu as pltpu
from jax.experimental.pallas import tpu_sc as plsc

B = 16384
D = 64
NE = 1000000
NR = 1000
L = 16
NC = 2
NS = 16
NW = NC * NS
TPT = B // NS
PPC = D // NC
RND = 128


def _plane_body(e12_hbm, rl_hbm,
                entT_hbm, relT_hbm,
                part_hbm,
                e12_v, rl_v,
                rel_v, htv, acc_v, spm0, spm1,
                ssem, gsem1, gsem2, rsem):
    cid = lax.axis_index("c")
    sid = lax.axis_index("s")
    tbase = sid * TPT
    c0 = cid * PPC

    pltpu.sync_copy(e12_hbm.at[pl.ds(2 * tbase, 2 * TPT)], e12_v)
    pltpu.sync_copy(rl_hbm.at[pl.ds(tbase, TPT)], rl_v)

    for g in range(TPT // L):
        acc_v[pl.ds(g * L, L)] = jnp.zeros((L,), jnp.float32)

    @pl.when(sid == 0)
    def _():
        pltpu.async_copy(entT_hbm.at[c0], spm0, ssem)
        pltpu.async_copy(entT_hbm.at[c0 + 1], spm1, ssem)

    def plane_iter(k, carry):
        for par in range(2):
            c = c0 + 2 * k + par
            spm = spm0 if par == 0 else spm1
            pltpu.sync_copy(relT_hbm.at[c], rel_v)
            @pl.when(sid == 0)
            def _():
                pltpu.make_async_copy(
                    entT_hbm.at[c], spm, ssem).wait()
            plsc.subcore_barrier()

            nr = TPT // RND
            pltpu.async_copy(
                spm.at[e12_v.at[pl.ds(0, 2 * RND)]], htv[0], gsem1)
            for rnd in range(nr):
                cur = rnd % 2
                if rnd + 1 < nr:
                    roff = (rnd + 1) * 2 * RND
                    pltpu.async_copy(
                        spm.at[e12_v.at[pl.ds(roff, 2 * RND)]],
                        htv[1 - cur], gsem1)
                pltpu.make_async_copy(
                    spm.at[e12_v.at[pl.ds(rnd * 2 * RND, 2 * RND)]],
                    htv[cur], gsem1).wait()
                if rnd == nr - 1:
                    plsc.subcore_barrier()
                    @pl.when(jnp.logical_and(sid == 0, k < PPC // 2 - 1))
                    def _():
                        pltpu.async_copy(entT_hbm.at[c + 2], spm, ssem)
                for g in range(RND // L):
                    hsl = pl.ds(g * L, L)
                    tsl = pl.ds(RND + g * L, L)
                    sl = pl.ds(rnd * RND + g * L, L)
                    h = htv[cur][hsl]
                    t = htv[cur][tsl]
                    r = plsc.load_gather(rel_v, [rl_v[sl]])
                    acc_v[sl] = acc_v[sl] + jnp.abs(h + r - t)
        return carry

    lax.fori_loop(0, PPC // 2, plane_iter, 0)

    pltpu.sync_copy(acc_v, part_hbm.at[pl.ds(cid * B + tbase, TPT)])


def _sig_body(part_hbm, gam_hbm, out_hbm, p0_v, p1_v, out_v, gam_v):
    wid = lax.axis_index("s") * NC + lax.axis_index("c")
    base = wid * (B // NW)
    n = B // NW
    pltpu.sync_copy(part_hbm.at[pl.ds(base, n)], p0_v)
    pltpu.sync_copy(part_hbm.at[pl.ds(B + base, n)], p1_v)
    pltpu.sync_copy(gam_hbm, gam_v)
    gam = gam_v[...]
    for g in range(n // L):
        sl = pl.ds(g * L, L)
        score = gam - p0_v[sl] - p1_v[sl]
        out_v[sl] = 1.0 / (1.0 + jnp.exp(-score))
    pltpu.sync_copy(out_v, out_hbm.at[pl.ds(base, n)])


@jax.jit
def _transe_call(e12, rl, entT, relT, gam_vec):
    mesh = plsc.VectorSubcoreMesh(core_axis_name="c", subcore_axis_name="s")
    fa = pl.kernel(
        _plane_body,
        mesh=mesh,
        compiler_params=pltpu.CompilerParams(
            needs_layout_passes=False, internal_scratch_in_bytes=0),
        out_type=jax.ShapeDtypeStruct((2 * B,), jnp.float32),
        scratch_types=[
            pltpu.VMEM((2 * TPT,), jnp.int32),
            pltpu.VMEM((TPT,), jnp.int32),
            pltpu.VMEM((NR,), jnp.float32),
            [pltpu.VMEM((2 * RND,), jnp.float32),
             pltpu.VMEM((2 * RND,), jnp.float32)],
            pltpu.VMEM((TPT,), jnp.float32),
            pltpu.VMEM_SHARED((NE,), jnp.float32),
            pltpu.VMEM_SHARED((NE,), jnp.float32),
            pltpu.SemaphoreType.DMA,
            pltpu.SemaphoreType.DMA,
            pltpu.SemaphoreType.DMA,
            pltpu.SemaphoreType.DMA,
        ],
    )
    part = fa(e12, rl, entT, relT)
    fb = pl.kernel(
        _sig_body,
        mesh=mesh,
        compiler_params=pltpu.CompilerParams(needs_layout_passes=False),
        out_type=jax.ShapeDtypeStruct((B,), jnp.float32),
        scratch_types=[
            pltpu.VMEM((B // NW,), jnp.float32),
            pltpu.VMEM((B // NW,), jnp.float32),
            pltpu.VMEM((B // NW,), jnp.float32),
            pltpu.VMEM((L,), jnp.float32),
        ],
    )
    return fb(part, gam_vec)


def kernel(e1_idx, rel_idx, e2_idx, emb_ent_real, emb_rel_real, gamma):
    e1 = e1_idx.astype(jnp.int32)
    e2 = e2_idx.astype(jnp.int32)
    rl = rel_idx.astype(jnp.int32)
    e1r = e1.reshape(NS, TPT // RND, 1, RND)
    e2r = e2.reshape(NS, TPT // RND, 1, RND)
    e12 = jnp.concatenate([e1r, e2r], axis=2).reshape(2 * B)
    gam_vec = jnp.full((L,), gamma, jnp.float32)
    return _transe_call(e12, rl, emb_ent_real.T, emb_rel_real.T, gam_vec)
